# baseline (device time: 371339 ns/iter reference)
import jax
import jax.numpy as jnp
from jax import lax
from jax.experimental import pallas as pl
from jax.experimental.pallas import tpu as pltpu

N_DEV = 8
SUB = 4


def kernel(x):
    m_per, n = x.shape
    m_half = m_per // 2
    m_sub = m_half // SUB

    def body(x_ref, out_ref, stage_ref, local_sem,
             send_cw, recv_cw, send_ccw, recv_ccw):
        my = lax.axis_index("i")
        left = (my - 1) % N_DEV
        right = (my + 1) % N_DEV

        def row_cw(o, j):
            return o * m_per + j * m_sub

        def row_ccw(o, j):
            return o * m_per + m_half + j * m_sub

        barrier_sem = pltpu.get_barrier_semaphore()
        for nbr in (left, right):
            pl.semaphore_signal(
                barrier_sem, inc=1,
                device_id=(nbr,), device_id_type=pl.DeviceIdType.MESH,
            )
        pl.semaphore_wait(barrier_sem, 2)

        stage_ref[...] = x_ref[...].astype(jnp.bfloat16)
        cp = pltpu.make_async_copy(
            stage_ref, out_ref.at[pl.ds(my * m_per, m_per)], local_sem
        )
        cp.start()

        def send_desc(h, j, direction):
            if direction == "cw":
                o = (my - h) % N_DEV
                dst = out_ref.at[pl.ds(row_cw(o, j), m_sub)]
                src = (stage_ref.at[pl.ds(j * m_sub, m_sub)] if h == 0
                       else out_ref.at[pl.ds(row_cw(o, j), m_sub)])
                sem, dev = send_cw, right
            else:
                o = (my + h) % N_DEV
                dst = out_ref.at[pl.ds(row_ccw(o, j), m_sub)]
                src = (stage_ref.at[pl.ds(m_half + j * m_sub, m_sub)] if h == 0
                       else out_ref.at[pl.ds(row_ccw(o, j), m_sub)])
                sem, dev = send_ccw, left
            return pltpu.make_async_remote_copy(
                src_ref=src, dst_ref=dst,
                send_sem=sem.at[h, j], recv_sem=(recv_cw if direction == "cw"
                                                else recv_ccw).at[h, j],
                device_id=(dev,), device_id_type=pl.DeviceIdType.MESH,
            )

        def recv_desc(h, j, direction):
            if direction == "cw":
                o = (my - h - 1) % N_DEV
                dst = out_ref.at[pl.ds(row_cw(o, j), m_sub)]
                sem = recv_cw
            else:
                o = (my + h + 1) % N_DEV
                dst = out_ref.at[pl.ds(row_ccw(o, j), m_sub)]
                sem = recv_ccw
            return pltpu.make_async_remote_copy(
                src_ref=dst, dst_ref=dst,
                send_sem=(send_cw if direction == "cw" else send_ccw).at[h, j],
                recv_sem=sem.at[h, j],
                device_id=(left if direction == "cw" else right,),
                device_id_type=pl.DeviceIdType.MESH,
            )

        pending = []
        for j in range(SUB):
            for d in ("cw", "ccw"):
                s = send_desc(0, j, d)
                s.start()
                pending.append(s)
        for h in range(N_DEV - 1):
            for j in range(SUB):
                for d in ("cw", "ccw"):
                    recv_desc(h, j, d).wait_recv()
                    if h < N_DEV - 2:
                        s = send_desc(h + 1, j, d)
                        s.start()
                        pending.append(s)
        for s in pending:
            s.wait_send()
        cp.wait()

    return pl.pallas_call(
        body,
        out_shape=jax.ShapeDtypeStruct((N_DEV * m_per, n), jnp.bfloat16),
        in_specs=[pl.BlockSpec(memory_space=pltpu.VMEM)],
        out_specs=pl.BlockSpec(memory_space=pl.ANY),
        scratch_shapes=[
            pltpu.VMEM((m_per, n), jnp.bfloat16),
            pltpu.SemaphoreType.DMA,
            pltpu.SemaphoreType.DMA((N_DEV - 1, SUB)),
            pltpu.SemaphoreType.DMA((N_DEV - 1, SUB)),
            pltpu.SemaphoreType.DMA((N_DEV - 1, SUB)),
            pltpu.SemaphoreType.DMA((N_DEV - 1, SUB)),
        ],
        compiler_params=pltpu.CompilerParams(collective_id=0),
    )(x)


# device time: 270195 ns/iter; 1.3743x vs baseline; 1.3743x over previous
import jax
import jax.numpy as jnp
from jax import lax
from jax.experimental import pallas as pl
from jax.experimental.pallas import tpu as pltpu

N_DEV = 8

X, Y, Z = 1, 3, 4

THIRDS = (
    (0, 1376, (X, Y, Z)),
    (1376, 1360, (Y, Z, X)),
    (2736, 1360, (Z, X, Y)),
)


def kernel(x):
    m_per, n = x.shape
    assert m_per == 4096

    def body(x_ref, out_ref, stage_ref, local_sem, send_sems, recv_sems):
        my = lax.axis_index("i")

        barrier_sem = pltpu.get_barrier_semaphore()
        for mask in (X, Y, Z):
            pl.semaphore_signal(
                barrier_sem, inc=1,
                device_id=(my ^ mask,), device_id_type=pl.DeviceIdType.MESH,
            )
        pl.semaphore_wait(barrier_sem, 3)

        stage_ref[...] = x_ref[...].astype(jnp.bfloat16)
        cp = pltpu.make_async_copy(
            stage_ref, out_ref.at[pl.ds(my * m_per, m_per)], local_sem
        )
        cp.start()

        def held_origins(dims, k):
            acc = [0]
            for e in dims[:k]:
                acc = acc + [o ^ e for o in acc]
            return acc

        def phase_sends(t, k):
            r0, mt, dims = THIRDS[t]
            base = (1 << k) - 1
            partner = my ^ dims[k]
            descs = []
            for j, rel in enumerate(held_origins(dims, k)):
                o = my ^ rel
                dst = out_ref.at[pl.ds(o * m_per + r0, mt)]
                src = (stage_ref.at[pl.ds(r0, mt)] if rel == 0
                       else out_ref.at[pl.ds(o * m_per + r0, mt)])
                d = pltpu.make_async_remote_copy(
                    src_ref=src, dst_ref=dst,
                    send_sem=send_sems.at[t, base + j],
                    recv_sem=recv_sems.at[t, base + j],
                    device_id=(partner,),
                    device_id_type=pl.DeviceIdType.MESH,
                )
                d.start()
                descs.append(d)
            return descs

        def phase_recv_wait(t, k):
            r0, mt, dims = THIRDS[t]
            base = (1 << k) - 1
            partner = my ^ dims[k]
            for j, rel in enumerate(held_origins(dims, k)):
                o = partner ^ rel
                dst = out_ref.at[pl.ds(o * m_per + r0, mt)]
                pltpu.make_async_remote_copy(
                    src_ref=dst, dst_ref=dst,
                    send_sem=send_sems.at[t, base + j],
                    recv_sem=recv_sems.at[t, base + j],
                    device_id=(partner,),
                    device_id_type=pl.DeviceIdType.MESH,
                ).wait_recv()

        pending = []
        for t in range(3):
            pending += phase_sends(t, 0)
        for k in range(1, 3):
            for t in range(3):
                phase_recv_wait(t, k - 1)
                pending += phase_sends(t, k)
        for t in range(3):
            phase_recv_wait(t, 2)
        for d in pending:
            d.wait_send()
        cp.wait()

    return pl.pallas_call(
        body,
        out_shape=jax.ShapeDtypeStruct((N_DEV * m_per, n), jnp.bfloat16),
        in_specs=[pl.BlockSpec(memory_space=pltpu.VMEM)],
        out_specs=pl.BlockSpec(memory_space=pl.ANY),
        scratch_shapes=[
            pltpu.VMEM((m_per, n), jnp.bfloat16),
            pltpu.SemaphoreType.DMA,
            pltpu.SemaphoreType.DMA((3, 7)),
            pltpu.SemaphoreType.DMA((3, 7)),
        ],
        compiler_params=pltpu.CompilerParams(collective_id=0),
    )(x)


# device time: 270095 ns/iter; 1.3748x vs baseline; 1.0004x over previous
import jax
import jax.numpy as jnp
from jax import lax
from jax.experimental import pallas as pl
from jax.experimental.pallas import tpu as pltpu

N_DEV = 8

X, Y, Z = 1, 3, 4

THIRDS = (
    (0, 1376, (X, Y, Z)),
    (1376, 1360, (Y, Z, X)),
    (2736, 1360, (Z, X, Y)),
)
MT_MAX = 1376


def kernel(x):
    m_per, n = x.shape
    assert m_per == 4096

    def body(x_ref, out_ref, stage_ref, comm_ref,
             cp_sem, fwd_sems, send_sems, recv_sems):
        my = lax.axis_index("i")

        barrier_sem = pltpu.get_barrier_semaphore()
        for mask in (X, Y, Z):
            pl.semaphore_signal(
                barrier_sem, inc=1,
                device_id=(my ^ mask,), device_id_type=pl.DeviceIdType.MESH,
            )
        pl.semaphore_wait(barrier_sem, 3)

        stage_ref[...] = x_ref[...].astype(jnp.bfloat16)
        cp = pltpu.make_async_copy(
            stage_ref, out_ref.at[pl.ds(my * m_per, m_per)], cp_sem
        )
        cp.start()

        def comm_slot(t, j, mt):
            return comm_ref.at[t, pl.ds(j * MT_MAX, mt)]

        def held_rels(dims, k):
            acc = [0]
            for e in dims[:k]:
                acc = acc + [r ^ e for r in acc]
            return acc

        def phase_sends(t, k):
            r0, mt, dims = THIRDS[t]
            base = (1 << k) - 1
            partner = my ^ dims[k]
            descs = []
            for j, rel in enumerate(held_rels(dims, k)):
                src = (stage_ref.at[pl.ds(r0, mt)] if j == 0
                       else comm_slot(t, j - 1, mt))
                if k < 2:
                    dst = comm_slot(t, base + j, mt)
                else:
                    dst = out_ref.at[pl.ds((my ^ rel) * m_per + r0, mt)]
                d = pltpu.make_async_remote_copy(
                    src_ref=src, dst_ref=dst,
                    send_sem=send_sems.at[t, base + j],
                    recv_sem=recv_sems.at[t, base + j],
                    device_id=(partner,),
                    device_id_type=pl.DeviceIdType.MESH,
                )
                d.start()
                descs.append(d)
            return descs

        def phase_recv_wait(t, k):
            r0, mt, dims = THIRDS[t]
            base = (1 << k) - 1
            partner = my ^ dims[k]
            drains = []
            for j, rel in enumerate(held_rels(dims, k)):
                if k < 2:
                    dst = comm_slot(t, base + j, mt)
                else:
                    dst = out_ref.at[pl.ds((partner ^ rel) * m_per + r0, mt)]
                pltpu.make_async_remote_copy(
                    src_ref=dst, dst_ref=dst,
                    send_sem=send_sems.at[t, base + j],
                    recv_sem=recv_sems.at[t, base + j],
                    device_id=(partner,),
                    device_id_type=pl.DeviceIdType.MESH,
                ).wait_recv()
                if k < 2:
                    d = pltpu.make_async_copy(
                        comm_slot(t, base + j, mt),
                        out_ref.at[pl.ds((partner ^ rel) * m_per + r0, mt)],
                        fwd_sems.at[t, base + j],
                    )
                    d.start()
                    drains.append(d)
            return drains

        pending, drains = [], []
        for t in range(3):
            pending += phase_sends(t, 0)
        for k in range(1, 3):
            for t in range(3):
                drains += phase_recv_wait(t, k - 1)
                pending += phase_sends(t, k)
        for t in range(3):
            phase_recv_wait(t, 2)
        for d in pending:
            d.wait_send()
        for d in drains:
            d.wait()
        cp.wait()

    return pl.pallas_call(
        body,
        out_shape=jax.ShapeDtypeStruct((N_DEV * m_per, n), jnp.bfloat16),
        in_specs=[pl.BlockSpec(memory_space=pltpu.VMEM)],
        out_specs=pl.BlockSpec(memory_space=pl.ANY),
        scratch_shapes=[
            pltpu.VMEM((m_per, n), jnp.bfloat16),
            pltpu.VMEM((3, 3 * MT_MAX, n), jnp.bfloat16),
            pltpu.SemaphoreType.DMA,
            pltpu.SemaphoreType.DMA((3, 3)),
            pltpu.SemaphoreType.DMA((3, 7)),
            pltpu.SemaphoreType.DMA((3, 7)),
        ],
        compiler_params=pltpu.CompilerParams(
            collective_id=0,
            vmem_limit_bytes=52 * 1024 * 1024,
        ),
    )(x)
